# baseline (speedup 1.0000x reference)
import functools

import jax
import jax.numpy as jnp
from jax import lax
from jax.experimental import pallas as pl
from jax.experimental.pallas import tpu as pltpu
from jax.experimental.pallas import tpu_sc as plsc

N = 10000
E = 320000
D = 128
H = 128
C = 16

NC = 2
NS = 16
NW = NC * NS
EPT = E // NW
TROWS = N // NS
DEGPAD = NS * 640
ZDEG = DEGPAD // NS

_mesh = lambda: plsc.VectorSubcoreMesh(
    core_axis_name="c", subcore_axis_name="s", num_cores=NC, num_subcores=NS)


def _zero_vmem_2d(buf, rows, cols):
    z16 = jnp.zeros((16,), jnp.float32)

    @pl.loop(0, rows)
    def _(i):
        for j in range(cols // 16):
            buf[i, pl.ds(j * 16, 16)] = z16


def _deg_kernel_body(ei_hbm, out_hbm, didx, ones_v, zbuf, agg, sem):
    cb = 128
    nch = EPT // cb
    tail = EPT - nch * cb
    ci = lax.axis_index("c")
    si = lax.axis_index("s")
    base = (ci * NS + si) * EPT
    pltpu.sync_copy(ei_hbm.at[1, pl.ds(base, EPT)], didx)

    for r in range(cb // 16):
        ones_v[pl.ds(r * 16, 16)] = jnp.ones((16,), jnp.float32)

    @pl.loop(0, ZDEG // 16)
    def _(i):
        zbuf[pl.ds(i * 16, 16)] = jnp.zeros((16,), jnp.float32)

    pltpu.sync_copy(zbuf, agg.at[pl.ds(si * ZDEG, ZDEG)])
    plsc.subcore_barrier()

    K = 26

    @pl.loop(0, nch // K)
    def _(g):
        @pl.loop(0, K)
        def _(j):
            pltpu.make_async_copy(
                ones_v, agg.at[didx.at[pl.ds((g * K + j) * cb, cb)]], sem) \
                .start(add=True)

        @pl.loop(0, K)
        def _(j):
            pltpu.make_async_copy(
                ones_v, agg.at[didx.at[pl.ds((g * K + j) * cb, cb)]], sem).wait()

    if tail:
        pltpu.sync_copy(ones_v.at[pl.ds(0, tail)],
                        agg.at[didx.at[pl.ds(nch * cb, tail)]], add=True)

    plsc.subcore_barrier()
    pltpu.sync_copy(agg.at[pl.ds(si * ZDEG, ZDEG)],
                    out_hbm.at[pl.ds(ci * DEGPAD + si * ZDEG, ZDEG)])


def _make_deg():
    return pl.kernel(
        _deg_kernel_body,
        out_type=jax.ShapeDtypeStruct((NC * DEGPAD,), jnp.float32),
        mesh=_mesh(),
        scratch_types=[
            pltpu.VMEM((EPT,), jnp.int32),
            pltpu.VMEM((128,), jnp.float32),
            pltpu.VMEM((ZDEG,), jnp.float32),
            pltpu.VMEM_SHARED((DEGPAD,), jnp.float32),
            pltpu.SemaphoreType.DMA,
        ],
        compiler_params=pltpu.CompilerParams(use_tc_tiling_on_sc=False),
    )


def _spmm_body(hdim, cb, nb, arows, wide, table_hbm, ei_hbm, out_hbm,
               sidx, didx, bufs, gsems, ssems, zsem, agg):
    nch = EPT // cb
    tail = EPT - nch * cb
    la = nb // 2
    trows = arows // NS
    ci = lax.axis_index("c")
    si = lax.axis_index("s")
    base = (ci * NS + si) * EPT
    pltpu.make_async_copy(ei_hbm.at[0, pl.ds(base, EPT)], sidx, gsems.at[0]) \
        .start()
    pltpu.make_async_copy(ei_hbm.at[1, pl.ds(base, EPT)], didx, gsems.at[1]) \
        .start()

    zrows = min(cb, trows)
    _zero_vmem_2d(bufs.at[0], zrows, hdim)
    nz = -(-trows // zrows)
    zstarts = [min(k * zrows, trows - zrows) for k in range(nz)]
    for r in zstarts:
        pltpu.make_async_copy(
            bufs.at[0].at[pl.ds(0, zrows)],
            agg.at[pl.ds(si * trows + r, zrows)], zsem).start()
    for r in zstarts:
        pltpu.make_async_copy(
            bufs.at[0].at[pl.ds(0, zrows)],
            agg.at[pl.ds(si * trows + r, zrows)], zsem).wait()
    pltpu.make_async_copy(ei_hbm.at[0, pl.ds(base, EPT)], sidx, gsems.at[0]) \
        .wait()
    pltpu.make_async_copy(ei_hbm.at[1, pl.ds(base, EPT)], didx, gsems.at[1]) \
        .wait()
    plsc.subcore_barrier()

    def gath(c, b):
        return pltpu.make_async_copy(
            table_hbm.at[sidx.at[pl.ds(c * cb, cb)]], bufs.at[b], gsems.at[b])

    def scat(c, b):
        return pltpu.make_async_copy(
            bufs.at[b], agg.at[didx.at[pl.ds(c * cb, cb)]], ssems.at[b])

    for c in range(la):
        gath(c, c).start()

    @pl.loop(0, nch + nb - 1 - ((nch - 1) % nb), step=nb)
    def _(g):
        for b in range(nb):
            c = g + b
            bn = (b + la) % nb

            @pl.when(jnp.logical_and(c >= la, c + la < nch))
            def _():
                scat(c - la, bn).wait()

            @pl.when(c + la < nch)
            def _():
                gath(c + la, bn).start()

            @pl.when(c < nch)
            def _():
                gath(c, b).wait()
                scat(c, b).start(add=True)

    for c in range(nch - nb, nch):
        scat(c, c % nb).wait()

    if tail:
        t0 = nch * cb
        pltpu.sync_copy(table_hbm.at[sidx.at[pl.ds(t0, tail)]],
                        bufs.at[0].at[pl.ds(0, tail)])
        pltpu.sync_copy(bufs.at[0].at[pl.ds(0, tail)],
                        agg.at[didx.at[pl.ds(t0, tail)]], add=True)

    plsc.subcore_barrier()
    r0 = si * trows
    if wide:
        pltpu.sync_copy(agg.at[pl.ds(r0, trows), :],
                        out_hbm.at[pl.ds(ci * arows + r0, trows), pl.ds(0, hdim)])
    else:
        pltpu.sync_copy(agg.at[pl.ds(r0, trows)],
                        out_hbm.at[pl.ds(ci * arows + r0, trows)])


def _make_spmm(hdim, cb, nb, arows=N, wide=False):
    return pl.kernel(
        functools.partial(_spmm_body, hdim, cb, nb, arows, wide),
        out_type=jax.ShapeDtypeStruct(
            (NC * arows, 128) if wide else (NC * arows, hdim), jnp.float32),
        mesh=_mesh(),
        scratch_types=[
            pltpu.VMEM((EPT,), jnp.int32),
            pltpu.VMEM((EPT,), jnp.int32),
            pltpu.VMEM((nb, cb, hdim), jnp.float32),
            pltpu.SemaphoreType.DMA((nb,)),
            pltpu.SemaphoreType.DMA((nb,)),
            pltpu.SemaphoreType.DMA,
            pltpu.VMEM_SHARED((arows, hdim), jnp.float32),
        ],
        compiler_params=pltpu.CompilerParams(use_tc_tiling_on_sc=False),
    )


def _dinv_bcast(degp_ref, width):
    d2 = jnp.reshape(degp_ref[...], (NC, DEGPAD))
    deg = d2[0] + d2[1]
    safe = jnp.where(deg > 0.0, deg, 1.0)
    dinvr = jnp.where(deg > 0.0, lax.rsqrt(safe), 0.0)
    return lax.broadcast_in_dim(dinvr, (DEGPAD, width), (0,))[:N]


def _mm_body(x_ref, w_ref, h_ref):
    h_ref[...] = jnp.dot(x_ref[...], w_ref[...],
                         preferred_element_type=jnp.float32)


def _mm(x, W1):
    return pl.pallas_call(
        _mm_body,
        out_shape=jax.ShapeDtypeStruct((N, H), jnp.float32),
    )(x, W1)


def _tc1_body(degp_ref, h_ref, t1_ref):
    t1_ref[...] = h_ref[...] * _dinv_bcast(degp_ref, H)


def _tc1(degp, h):
    return pl.pallas_call(
        _tc1_body,
        out_shape=jax.ShapeDtypeStruct((N, H), jnp.float32),
    )(degp, h)


def _tc2_body(degp_ref, p0_ref, p1_ref, b1_ref, w_ref, t2_ref):
    db = _dinv_bcast(degp_ref, H)
    agg = p0_ref[...] + p1_ref[...]
    z = jnp.maximum(agg * db + b1_ref[...], 0.0)
    h2 = jnp.dot(z, w_ref[...], preferred_element_type=jnp.float32)
    t2_ref[...] = h2 * db[:, :C]


def _tc2(degp, p, b1, W2):
    return pl.pallas_call(
        _tc2_body,
        grid=(1,),
        in_specs=[
            pl.BlockSpec((NC * DEGPAD // 128, 128), lambda i: (0, 0)),
            pl.BlockSpec((N, H), lambda i: (0, 0)),
            pl.BlockSpec((N, H), lambda i: (1, 0)),
            pl.BlockSpec((1, H), lambda i: (0, 0)),
            pl.BlockSpec((H, C), lambda i: (0, 0)),
        ],
        out_specs=pl.BlockSpec((N, C), lambda i: (0, 0)),
        out_shape=jax.ShapeDtypeStruct((N, C), jnp.float32),
    )(degp, p, p, b1, W2)


_QROWS = 10240


def _tc3_body(degp_ref, q0_ref, q1_ref, b2_ref, out_ref):
    db = _dinv_bcast(degp_ref, C)
    qs = q0_ref[...][:N, :C] + q1_ref[...][:N, :C]
    out_ref[...] = qs * db + b2_ref[...]


def _tc3(degp, q, b2):
    return pl.pallas_call(
        _tc3_body,
        grid=(1,),
        in_specs=[
            pl.BlockSpec((NC * DEGPAD // 128, 128), lambda i: (0, 0)),
            pl.BlockSpec((_QROWS, 128), lambda i: (0, 0)),
            pl.BlockSpec((_QROWS, 128), lambda i: (1, 0)),
            pl.BlockSpec((1, C), lambda i: (0, 0)),
        ],
        out_specs=pl.BlockSpec((N, C), lambda i: (0, 0)),
        out_shape=jax.ShapeDtypeStruct((N, C), jnp.float32),
    )(degp, q, q, b2)


def kernel(x, edge_index, W1, b1, W2, b2):
    ei = edge_index.astype(jnp.int32)
    h1 = _mm(x, W1)
    degf = _make_deg()(ei)
    degp = degf.reshape(NC * DEGPAD // 128, 128)
    t1 = _tc1(degp, h1)
    p = _make_spmm(H, 48, 4)(t1, ei)
    t2 = _tc2(degp, p, b1.reshape(1, H), W2)
    q = _make_spmm(C, 1000, 4, arows=_QROWS, wide=True)(t2, ei)
    return _tc3(degp, q, b2.reshape(1, C))


# speedup vs baseline: 1.0035x; 1.0035x over previous
import functools

import jax
import jax.numpy as jnp
from jax import lax
from jax.experimental import pallas as pl
from jax.experimental.pallas import tpu as pltpu
from jax.experimental.pallas import tpu_sc as plsc

N = 10000
E = 320000
D = 128
H = 128
C = 16

NC = 2
NS = 16
NW = NC * NS
EPT = E // NW
TROWS = N // NS
DEGPAD = NS * 640
ZDEG = DEGPAD // NS

_mesh = lambda: plsc.VectorSubcoreMesh(
    core_axis_name="c", subcore_axis_name="s", num_cores=NC, num_subcores=NS)


def _zero_vmem_2d(buf, rows, cols):
    z16 = jnp.zeros((16,), jnp.float32)

    @pl.loop(0, rows)
    def _(i):
        for j in range(cols // 16):
            buf[i, pl.ds(j * 16, 16)] = z16


def _deg_kernel_body(ei_hbm, out_hbm, didx, ones_v, zbuf, agg, sem):
    cb = 128
    nch = EPT // cb
    tail = EPT - nch * cb
    ci = lax.axis_index("c")
    si = lax.axis_index("s")
    base = (ci * NS + si) * EPT
    pltpu.sync_copy(ei_hbm.at[1, pl.ds(base, EPT)], didx)

    for r in range(cb // 16):
        ones_v[pl.ds(r * 16, 16)] = jnp.ones((16,), jnp.float32)

    @pl.loop(0, ZDEG // 16)
    def _(i):
        zbuf[pl.ds(i * 16, 16)] = jnp.zeros((16,), jnp.float32)

    pltpu.sync_copy(zbuf, agg.at[pl.ds(si * ZDEG, ZDEG)])
    plsc.subcore_barrier()

    K = 26

    @pl.loop(0, nch // K)
    def _(g):
        @pl.loop(0, K)
        def _(j):
            pltpu.make_async_copy(
                ones_v, agg.at[didx.at[pl.ds((g * K + j) * cb, cb)]], sem) \
                .start(add=True)

        @pl.loop(0, K)
        def _(j):
            pltpu.make_async_copy(
                ones_v, agg.at[didx.at[pl.ds((g * K + j) * cb, cb)]], sem).wait()

    if tail:
        pltpu.sync_copy(ones_v.at[pl.ds(0, tail)],
                        agg.at[didx.at[pl.ds(nch * cb, tail)]], add=True)

    plsc.subcore_barrier()
    pltpu.sync_copy(agg.at[pl.ds(si * ZDEG, ZDEG)],
                    out_hbm.at[pl.ds(ci * DEGPAD + si * ZDEG, ZDEG)])


def _make_deg():
    return pl.kernel(
        _deg_kernel_body,
        out_type=jax.ShapeDtypeStruct((NC * DEGPAD,), jnp.float32),
        mesh=_mesh(),
        scratch_types=[
            pltpu.VMEM((EPT,), jnp.int32),
            pltpu.VMEM((128,), jnp.float32),
            pltpu.VMEM((ZDEG,), jnp.float32),
            pltpu.VMEM_SHARED((DEGPAD,), jnp.float32),
            pltpu.SemaphoreType.DMA,
        ],
        compiler_params=pltpu.CompilerParams(use_tc_tiling_on_sc=False),
    )


def _spmm_body(hdim, cb, nb, arows, wide, table_hbm, ei_hbm, out_hbm,
               sidx, didx, bufs, gsems, ssems, zsem, agg):
    nch = EPT // cb
    tail = EPT - nch * cb
    la = nb // 2
    trows = arows // NS
    ci = lax.axis_index("c")
    si = lax.axis_index("s")
    base = (ci * NS + si) * EPT
    pltpu.make_async_copy(ei_hbm.at[0, pl.ds(base, EPT)], sidx, gsems.at[0]) \
        .start()
    pltpu.make_async_copy(ei_hbm.at[1, pl.ds(base, EPT)], didx, gsems.at[1]) \
        .start()

    zrows = min(cb, trows)
    _zero_vmem_2d(bufs.at[0], zrows, hdim)
    nz = -(-trows // zrows)
    zstarts = [min(k * zrows, trows - zrows) for k in range(nz)]
    for r in zstarts:
        pltpu.make_async_copy(
            bufs.at[0].at[pl.ds(0, zrows)],
            agg.at[pl.ds(si * trows + r, zrows)], zsem).start()
    for r in zstarts:
        pltpu.make_async_copy(
            bufs.at[0].at[pl.ds(0, zrows)],
            agg.at[pl.ds(si * trows + r, zrows)], zsem).wait()
    pltpu.make_async_copy(ei_hbm.at[0, pl.ds(base, EPT)], sidx, gsems.at[0]) \
        .wait()
    pltpu.make_async_copy(ei_hbm.at[1, pl.ds(base, EPT)], didx, gsems.at[1]) \
        .wait()
    plsc.subcore_barrier()

    def gath(c, b):
        return pltpu.make_async_copy(
            table_hbm.at[sidx.at[pl.ds(c * cb, cb)]], bufs.at[b], gsems.at[b])

    def scat(c, b):
        return pltpu.make_async_copy(
            bufs.at[b], agg.at[didx.at[pl.ds(c * cb, cb)]], ssems.at[b])

    for c in range(la):
        gath(c, c).start()

    @pl.loop(0, nch + nb - 1 - ((nch - 1) % nb), step=nb)
    def _(g):
        for b in range(nb):
            c = g + b
            bn = (b + la) % nb

            @pl.when(jnp.logical_and(c >= la, c + la < nch))
            def _():
                scat(c - la, bn).wait()

            @pl.when(c + la < nch)
            def _():
                gath(c + la, bn).start()

            @pl.when(c < nch)
            def _():
                gath(c, b).wait()
                scat(c, b).start(add=True)

    for c in range(nch - nb, nch):
        scat(c, c % nb).wait()

    if tail:
        t0 = nch * cb
        pltpu.sync_copy(table_hbm.at[sidx.at[pl.ds(t0, tail)]],
                        bufs.at[0].at[pl.ds(0, tail)])
        pltpu.sync_copy(bufs.at[0].at[pl.ds(0, tail)],
                        agg.at[didx.at[pl.ds(t0, tail)]], add=True)

    plsc.subcore_barrier()
    r0 = si * trows
    if wide:
        pltpu.sync_copy(agg.at[pl.ds(r0, trows), :],
                        out_hbm.at[pl.ds(ci * arows + r0, trows), pl.ds(0, hdim)])
    else:
        pltpu.sync_copy(agg.at[pl.ds(r0, trows)],
                        out_hbm.at[pl.ds(ci * arows + r0, trows)])


def _make_spmm(hdim, cb, nb, arows=N, wide=False):
    return pl.kernel(
        functools.partial(_spmm_body, hdim, cb, nb, arows, wide),
        out_type=jax.ShapeDtypeStruct(
            (NC * arows, 128) if wide else (NC * arows, hdim), jnp.float32),
        mesh=_mesh(),
        scratch_types=[
            pltpu.VMEM((EPT,), jnp.int32),
            pltpu.VMEM((EPT,), jnp.int32),
            pltpu.VMEM((nb, cb, hdim), jnp.float32),
            pltpu.SemaphoreType.DMA((nb,)),
            pltpu.SemaphoreType.DMA((nb,)),
            pltpu.SemaphoreType.DMA,
            pltpu.VMEM_SHARED((arows, hdim), jnp.float32),
        ],
        compiler_params=pltpu.CompilerParams(use_tc_tiling_on_sc=False),
    )


def _dinv_bcast(degp_ref, width):
    d2 = jnp.reshape(degp_ref[...], (NC, DEGPAD))
    deg = d2[0] + d2[1]
    safe = jnp.where(deg > 0.0, deg, 1.0)
    dinvr = jnp.where(deg > 0.0, lax.rsqrt(safe), 0.0)
    return lax.broadcast_in_dim(dinvr, (DEGPAD, width), (0,))[:N]


def _tc1_body(degp_ref, x_ref, w_ref, t1_ref):
    db = _dinv_bcast(degp_ref, H)
    h = jnp.dot(x_ref[...], w_ref[...], preferred_element_type=jnp.float32)
    t1_ref[...] = h * db


def _tc1(degp, x, W1):
    return pl.pallas_call(
        _tc1_body,
        out_shape=jax.ShapeDtypeStruct((N, H), jnp.float32),
    )(degp, x, W1)


def _tc2_body(degp_ref, p0_ref, p1_ref, b1_ref, w_ref, t2_ref):
    db = _dinv_bcast(degp_ref, H)
    agg = p0_ref[...] + p1_ref[...]
    z = jnp.maximum(agg * db + b1_ref[...], 0.0)
    h2 = jnp.dot(z, w_ref[...], preferred_element_type=jnp.float32)
    t2_ref[...] = h2 * db[:, :C]


def _tc2(degp, p, b1, W2):
    return pl.pallas_call(
        _tc2_body,
        grid=(1,),
        in_specs=[
            pl.BlockSpec((NC * DEGPAD // 128, 128), lambda i: (0, 0)),
            pl.BlockSpec((N, H), lambda i: (0, 0)),
            pl.BlockSpec((N, H), lambda i: (1, 0)),
            pl.BlockSpec((1, H), lambda i: (0, 0)),
            pl.BlockSpec((H, C), lambda i: (0, 0)),
        ],
        out_specs=pl.BlockSpec((N, C), lambda i: (0, 0)),
        out_shape=jax.ShapeDtypeStruct((N, C), jnp.float32),
    )(degp, p, p, b1, W2)


_QROWS = 10240


def _tc3_body(degp_ref, q0_ref, q1_ref, b2_ref, out_ref):
    db = _dinv_bcast(degp_ref, C)
    qs = q0_ref[...][:N, :C] + q1_ref[...][:N, :C]
    out_ref[...] = qs * db + b2_ref[...]


def _tc3(degp, q, b2):
    return pl.pallas_call(
        _tc3_body,
        grid=(1,),
        in_specs=[
            pl.BlockSpec((NC * DEGPAD // 128, 128), lambda i: (0, 0)),
            pl.BlockSpec((_QROWS, 128), lambda i: (0, 0)),
            pl.BlockSpec((_QROWS, 128), lambda i: (1, 0)),
            pl.BlockSpec((1, C), lambda i: (0, 0)),
        ],
        out_specs=pl.BlockSpec((N, C), lambda i: (0, 0)),
        out_shape=jax.ShapeDtypeStruct((N, C), jnp.float32),
    )(degp, q, q, b2)


def kernel(x, edge_index, W1, b1, W2, b2):
    ei = edge_index.astype(jnp.int32)
    degf = _make_deg()(ei)
    degp = degf.reshape(NC * DEGPAD // 128, 128)
    t1 = _tc1(degp, x, W1)
    p = _make_spmm(H, 48, 4)(t1, ei)
    t2 = _tc2(degp, p, b1.reshape(1, H), W2)
    q = _make_spmm(C, 1000, 4, arows=_QROWS, wide=True)(t2, ei)
    return _tc3(degp, q, b2.reshape(1, C))
